# baseline (speedup 1.0000x reference)
import functools

import numpy as np

import jax
import jax.numpy as jnp
from jax.experimental import pallas as pl
from jax.experimental.pallas import tpu as pltpu

_WINDOW = 4
_TAPS = 2 * _WINDOW + 1
_GRP = 16
_ALPHA = 0.2
_HEADS = 8
_NX = 64
_NM1 = 64
_NM2 = 64
_NTXT = _NX + _NM1 + _NM2
_N1 = _NM1 + _NX + _NX + _NM2
_N2 = _NX + _NM2
_BLK = 8


def _expand_mat():
    m = np.zeros((_TAPS * _HEADS, _HEADS * _GRP), np.float32)
    for t in range(_TAPS):
        for h in range(_HEADS):
            m[t * _HEADS + h, h * _GRP + t] = 1.0
    return m


def _group_sum_mat():
    m = np.zeros((_HEADS * _GRP, _HEADS * _GRP), np.float32)
    for h in range(_HEADS):
        m[h * _GRP:(h + 1) * _GRP, h * _GRP:(h + 1) * _GRP] = 1.0
    return m


def _bcast_mat(dh):
    m = np.zeros((_TAPS, _HEADS * _GRP, _HEADS * dh), np.float32)
    for t in range(_TAPS):
        for h in range(_HEADS):
            m[t, h * _GRP + t, h * dh:(h + 1) * dh] = 1.0
    return m


def _tap_mask(n, blk):
    m = np.full((n, _HEADS * _GRP), -1e9, np.float32)
    for t in range(_TAPS):
        o = t - _WINDOW
        for h in range(_HEADS):
            lane = h * _GRP + t
            lo, hi = max(0, -o), min(n, n - o)
            m[lo:hi, lane] = 0.0
    return np.tile(m, (blk, 1))


def _shift_rows(x, o, zeros_row):
    if o == 0:
        return x
    n = x.shape[0]
    z = jnp.broadcast_to(zeros_row, (abs(o), x.shape[1]))
    if o > 0:
        return jnp.concatenate([x[o:], z], axis=0)
    return jnp.concatenate([z, x[:n + o]], axis=0)


def _gat_banded(x, w, ablk, exp_ref, bd_ref, bc_ref, mask, *, dh, alpha):
    M = x.shape[0]
    wh = jnp.dot(x, w, preferred_element_type=jnp.float32)
    ft = jnp.einsum('kh,mk->hm', ablk, wh,
                    preferred_element_type=jnp.float32)
    f_src = ft[:_HEADS]
    f_dst = ft[_HEADS:]

    def lroll(xx, o):
        k = o % M
        if k == 0:
            return xx
        return jnp.concatenate([xx[:, k:], xx[:, :k]], axis=1)

    s_t = jnp.concatenate(
        [f_src + lroll(f_dst, t - _WINDOW) for t in range(_TAPS)],
        axis=0)
    e = jnp.einsum('tm,tl->ml', s_t, exp_ref[...],
                   preferred_element_type=jnp.float32)
    e = jnp.where(e > 0, e, alpha * e)
    n = mask.shape[0]
    e = (e.reshape(M // n, n, _HEADS * _GRP) + mask[None]).reshape(M, _HEADS * _GRP)
    e = e - jnp.max(e, axis=-1, keepdims=True)
    p = jnp.exp(e)
    gs = jnp.dot(p, bd_ref[...], preferred_element_type=jnp.float32)
    attn = p * pl.reciprocal(gs, approx=True)
    zwide = jnp.zeros((1, wh.shape[1]), jnp.float32)
    out = jnp.zeros_like(wh)
    attn16 = attn.astype(jnp.bfloat16)
    for t in range(_TAPS):
        a_t = jnp.dot(attn16, bc_ref[t], preferred_element_type=jnp.float32)
        out = out + a_t * _shift_rows(wh, t - _WINDOW, zwide)
    return jnp.where(out > 0, out, jnp.exp(jnp.minimum(out, 0.0)) - 1.0)


def _model_kernel(text_ref, vis_ref, aud_ref,
                  wv_ref, bv_ref, wa_ref, ba_ref,
                  xw_ref, xa_ref, mx_ref,
                  m2w_ref, m2a_ref, mm2_ref,
                  m1w_ref, m1a_ref, mm1_ref,
                  exp_ref, bd_ref, bc_ref,
                  w1_ref, b1_ref, w2_ref, b2_ref,
                  erc_ref, ecpe_ref, o_ref,
                  *, dh, alpha):
    D = _HEADS * dh
    vr = jnp.dot(vis_ref[:, 0, :], wv_ref[...],
                 preferred_element_type=jnp.float32) + bv_ref[...]
    ar = jnp.dot(aud_ref[:, 0, :], wa_ref[...],
                 preferred_element_type=jnp.float32) + ba_ref[...]
    fused = 3.0 * text_ref[...] + (vr + ar)[:, None, :]

    x_h = fused[:, 0:_NX]
    m1_h = fused[:, _NX:_NX + _NM1]
    m2_h = fused[:, _NX + _NM1:_NTXT]

    g = functools.partial(_gat_banded, exp_ref=exp_ref, bd_ref=bd_ref,
                          bc_ref=bc_ref, dh=dh, alpha=alpha)
    x_o = g(x_h.reshape(_BLK * _NX, D), xw_ref[...], xa_ref[...],
            mask=mx_ref[...])
    x_o3 = x_o.reshape(_BLK, _NX, D)
    m2_in = jnp.concatenate([x_o3, m2_h], axis=1)
    m2_o = g(m2_in.reshape(_BLK * _N2, D), m2w_ref[...], m2a_ref[...],
             mask=mm2_ref[...])
    m2_o3 = m2_o.reshape(_BLK, _N2, D)
    m1_in = jnp.concatenate([m1_h, x_o3, m2_o3], axis=1)
    m1_o = g(m1_in.reshape(_BLK * _N1, D), m1w_ref[...], m1a_ref[...],
             mask=mm1_ref[...])

    rows3 = jnp.concatenate([m1_o.reshape(_BLK, _N1, D), m2_o3], axis=1)
    rows = rows3.reshape(_BLK * (_N1 + _N2), D)
    h = jnp.dot(rows, w1_ref[...], preferred_element_type=jnp.float32) + b1_ref[...]
    h = jnp.maximum(h, 0.0)
    z = jnp.dot(h, w2_ref[...], preferred_element_type=jnp.float32) + b2_ref[...]
    p = 1.0 / (1.0 + jnp.exp(-z))
    lse = jnp.log(jnp.sum(jnp.exp(p), axis=-1, keepdims=True))
    lbl = jnp.concatenate([erc_ref[...], ecpe_ref[...]],
                          axis=1).reshape(_BLK * (_N1 + _N2), 1)
    col = jax.lax.broadcasted_iota(jnp.int32, p.shape, 1)
    tgt = jnp.sum(jnp.where(col == lbl, p, 0.0), axis=-1, keepdims=True)
    row_loss = lse - tgt
    row = jax.lax.broadcasted_iota(jnp.int32, row_loss.shape, 0)
    is_m1 = (row % (_N1 + _N2)) < _N1
    o_ref[0, 0, 0] = jnp.sum(jnp.where(is_m1, row_loss, 0.0))
    o_ref[0, 0, 1] = jnp.sum(jnp.where(is_m1, 0.0, row_loss))


def kernel(causal_text, visual_feat, acoustic_feat, erc_label, ecpe_label,
           vis_w, vis_b, aud_w, aud_b, cls_w1, cls_b1, cls_w2, cls_b2,
           x_W, x_asrc, x_adst, m2_W, m2_asrc, m2_adst, m1_W, m1_asrc, m1_adst):
    B, S, D = causal_text.shape
    Dv = vis_w.shape[0]
    Da = aud_w.shape[0]
    C = cls_w2.shape[1]
    H = _HEADS
    dh = D // H
    nblk = B // _BLK

    def ablk(a_src, a_dst):
        eye = jnp.eye(H, dtype=jnp.float32)
        bs = (a_src[:, 0, :, None] * eye[:, None, :]).reshape(D, H)
        bd = (a_dst[:, 0, :, None] * eye[:, None, :]).reshape(D, H)
        return jnp.concatenate([bs, bd], axis=1)

    x_a = ablk(x_asrc, x_adst)
    m2_a = ablk(m2_asrc, m2_adst)
    m1_a = ablk(m1_asrc, m1_adst)

    vis3 = visual_feat.reshape(B, 1, Dv)
    aud3 = acoustic_feat.reshape(B, 1, Da)
    erc3 = erc_label.reshape(B, _N1, 1).astype(jnp.int32)
    ecpe3 = ecpe_label.reshape(B, _N2, 1).astype(jnp.int32)

    const = lambda shape: pl.BlockSpec(shape, lambda b: tuple(0 for _ in shape))
    _kernel_fn = functools.partial(_model_kernel, dh=dh, alpha=_ALPHA)
    out = pl.pallas_call(
        _kernel_fn,
        out_shape=jax.ShapeDtypeStruct((nblk, 1, 2), jnp.float32),
        grid=(nblk,),
        in_specs=[
            pl.BlockSpec((_BLK, _NTXT, D), lambda b: (b, 0, 0)),
            pl.BlockSpec((_BLK, 1, Dv), lambda b: (b, 0, 0)),
            pl.BlockSpec((_BLK, 1, Da), lambda b: (b, 0, 0)),
            const((Dv, D)), const((1, D)), const((Da, D)), const((1, D)),
            const((D, D)), const((D, 2 * H)), const((_NX, H * _GRP)),
            const((D, D)), const((D, 2 * H)), const((_N2, H * _GRP)),
            const((D, D)), const((D, 2 * H)), const((_N1, H * _GRP)),
            const((_TAPS * H, H * _GRP)), const((H * _GRP, H * _GRP)),
            const((_TAPS, H * _GRP, D)),
            const((D, D)), const((1, D)), const((D, C)), const((1, C)),
            pl.BlockSpec((_BLK, _N1, 1), lambda b: (b, 0, 0)),
            pl.BlockSpec((_BLK, _N2, 1), lambda b: (b, 0, 0)),
        ],
        out_specs=pl.BlockSpec((1, 1, 2), lambda b: (b, 0, 0),
                               memory_space=pltpu.MemorySpace.SMEM),
        compiler_params=pltpu.CompilerParams(
            dimension_semantics=("parallel",)),
    )(causal_text, vis3, aud3,
      vis_w, vis_b.reshape(1, D), aud_w, aud_b.reshape(1, D),
      x_W, x_a, jnp.asarray(_tap_mask(_NX, 1)),
      m2_W, m2_a, jnp.asarray(_tap_mask(_N2, 1)),
      m1_W, m1_a, jnp.asarray(_tap_mask(_N1, 1)),
      jnp.asarray(_expand_mat()), jnp.asarray(_group_sum_mat()),
      jnp.asarray(_bcast_mat(dh), jnp.bfloat16),
      cls_w1, cls_b1.reshape(1, D), cls_w2, cls_b2.reshape(1, C),
      erc3, ecpe3)

    emotion_loss = jnp.sum(out[:, 0, 0]) / float(B * _N1)
    cause_loss = jnp.sum(out[:, 0, 1]) / float(B * _N2)
    return emotion_loss, cause_loss


# speedup vs baseline: 1.0437x; 1.0437x over previous
import functools

import numpy as np

import jax
import jax.numpy as jnp
from jax.experimental import pallas as pl
from jax.experimental.pallas import tpu as pltpu

_WINDOW = 4
_TAPS = 2 * _WINDOW + 1
_GRP = 16
_ALPHA = 0.2
_HEADS = 8
_NX = 64
_NM1 = 64
_NM2 = 64
_NTXT = _NX + _NM1 + _NM2
_N1 = _NM1 + _NX + _NX + _NM2
_N2 = _NX + _NM2
_BLK = 16


def _expand_mat():
    m = np.zeros((_TAPS * _HEADS, _HEADS * _GRP), np.float32)
    for t in range(_TAPS):
        for h in range(_HEADS):
            m[t * _HEADS + h, h * _GRP + t] = 1.0
    return m


def _group_sum_mat():
    m = np.zeros((_HEADS * _GRP, _HEADS * _GRP), np.float32)
    for h in range(_HEADS):
        m[h * _GRP:(h + 1) * _GRP, h * _GRP:(h + 1) * _GRP] = 1.0
    return m


def _bcast_mat(dh):
    m = np.zeros((_TAPS, _HEADS * _GRP, _HEADS * dh), np.float32)
    for t in range(_TAPS):
        for h in range(_HEADS):
            m[t, h * _GRP + t, h * dh:(h + 1) * dh] = 1.0
    return m


def _tap_mask(n, blk):
    m = np.full((n, _HEADS * _GRP), -1e9, np.float32)
    for t in range(_TAPS):
        o = t - _WINDOW
        for h in range(_HEADS):
            lane = h * _GRP + t
            lo, hi = max(0, -o), min(n, n - o)
            m[lo:hi, lane] = 0.0
    return np.tile(m, (blk, 1))


def _shift_rows(x, o, zeros_row):
    if o == 0:
        return x
    n = x.shape[0]
    z = jnp.broadcast_to(zeros_row, (abs(o), x.shape[1]))
    if o > 0:
        return jnp.concatenate([x[o:], z], axis=0)
    return jnp.concatenate([z, x[:n + o]], axis=0)


def _gat_banded(x, w, ablk, exp_ref, bd_ref, bc_ref, mask, *, dh, alpha):
    M = x.shape[0]
    wh = jnp.dot(x, w, preferred_element_type=jnp.float32)
    ft = jnp.einsum('kh,mk->hm', ablk, wh,
                    preferred_element_type=jnp.float32)
    f_src = ft[:_HEADS]
    f_dst = ft[_HEADS:]

    def lroll(xx, o):
        k = o % M
        if k == 0:
            return xx
        return jnp.concatenate([xx[:, k:], xx[:, :k]], axis=1)

    s_t = jnp.concatenate(
        [f_src + lroll(f_dst, t - _WINDOW) for t in range(_TAPS)],
        axis=0)
    e = jnp.einsum('tm,tl->ml', s_t, exp_ref[...],
                   preferred_element_type=jnp.float32)
    e = jnp.where(e > 0, e, alpha * e)
    n = mask.shape[0]
    e = (e.reshape(M // n, n, _HEADS * _GRP) + mask[None]).reshape(M, _HEADS * _GRP)
    e = e - jnp.max(e, axis=-1, keepdims=True)
    p = jnp.exp(e)
    gs = jnp.dot(p, bd_ref[...], preferred_element_type=jnp.float32)
    attn = p * pl.reciprocal(gs, approx=True)
    zwide = jnp.zeros((1, wh.shape[1]), jnp.float32)
    out = jnp.zeros_like(wh)
    attn16 = attn.astype(jnp.bfloat16)
    for t in range(_TAPS):
        a_t = jnp.dot(attn16, bc_ref[t], preferred_element_type=jnp.float32)
        out = out + a_t * _shift_rows(wh, t - _WINDOW, zwide)
    return jnp.where(out > 0, out, jnp.exp(out) - 1.0)


def _model_kernel(text_ref, vis_ref, aud_ref,
                  wv_ref, bv_ref, wa_ref, ba_ref,
                  xw_ref, xa_ref, mx_ref,
                  m2w_ref, m2a_ref, mm2_ref,
                  m1w_ref, m1a_ref, mm1_ref,
                  exp_ref, bd_ref, bc_ref,
                  w1_ref, b1_ref, w2_ref, b2_ref,
                  erc_ref, ecpe_ref, o_ref,
                  *, dh, alpha):
    D = _HEADS * dh
    vr = jnp.dot(vis_ref[:, 0, :], wv_ref[...],
                 preferred_element_type=jnp.float32) + bv_ref[...]
    ar = jnp.dot(aud_ref[:, 0, :], wa_ref[...],
                 preferred_element_type=jnp.float32) + ba_ref[...]
    fused = 3.0 * text_ref[...] + (vr + ar)[:, None, :]

    x_h = fused[:, 0:_NX]
    m1_h = fused[:, _NX:_NX + _NM1]
    m2_h = fused[:, _NX + _NM1:_NTXT]

    g = functools.partial(_gat_banded, exp_ref=exp_ref, bd_ref=bd_ref,
                          bc_ref=bc_ref, dh=dh, alpha=alpha)
    x_o = g(x_h.reshape(_BLK * _NX, D), xw_ref[...], xa_ref[...],
            mask=mx_ref[...])
    x_o3 = x_o.reshape(_BLK, _NX, D)
    m2_in = jnp.concatenate([x_o3, m2_h], axis=1)
    m2_o = g(m2_in.reshape(_BLK * _N2, D), m2w_ref[...], m2a_ref[...],
             mask=mm2_ref[...])
    m2_o3 = m2_o.reshape(_BLK, _N2, D)
    m1_in = jnp.concatenate([m1_h, x_o3, m2_o3], axis=1)
    m1_o = g(m1_in.reshape(_BLK * _N1, D), m1w_ref[...], m1a_ref[...],
             mask=mm1_ref[...])

    rows3 = jnp.concatenate([m1_o.reshape(_BLK, _N1, D), m2_o3], axis=1)
    rows = rows3.reshape(_BLK * (_N1 + _N2), D)
    h = jnp.dot(rows, w1_ref[...], preferred_element_type=jnp.float32) + b1_ref[...]
    h = jnp.maximum(h, 0.0)
    z = jnp.dot(h, w2_ref[...], preferred_element_type=jnp.float32) + b2_ref[...]
    p = 1.0 / (1.0 + jnp.exp(-z))
    lse = jnp.log(jnp.sum(jnp.exp(p), axis=-1, keepdims=True))
    lbl = jnp.concatenate([erc_ref[...], ecpe_ref[...]],
                          axis=1).reshape(_BLK * (_N1 + _N2), 1)
    col = jax.lax.broadcasted_iota(jnp.int32, p.shape, 1)
    tgt = jnp.sum(jnp.where(col == lbl, p, 0.0), axis=-1, keepdims=True)
    row_loss = lse - tgt
    row = jax.lax.broadcasted_iota(jnp.int32, row_loss.shape, 0)
    is_m1 = (row % (_N1 + _N2)) < _N1
    o_ref[0, 0, 0] = jnp.sum(jnp.where(is_m1, row_loss, 0.0))
    o_ref[0, 0, 1] = jnp.sum(jnp.where(is_m1, 0.0, row_loss))


def kernel(causal_text, visual_feat, acoustic_feat, erc_label, ecpe_label,
           vis_w, vis_b, aud_w, aud_b, cls_w1, cls_b1, cls_w2, cls_b2,
           x_W, x_asrc, x_adst, m2_W, m2_asrc, m2_adst, m1_W, m1_asrc, m1_adst):
    B, S, D = causal_text.shape
    Dv = vis_w.shape[0]
    Da = aud_w.shape[0]
    C = cls_w2.shape[1]
    H = _HEADS
    dh = D // H
    nblk = B // _BLK

    def ablk(a_src, a_dst):
        eye = jnp.eye(H, dtype=jnp.float32)
        bs = (a_src[:, 0, :, None] * eye[:, None, :]).reshape(D, H)
        bd = (a_dst[:, 0, :, None] * eye[:, None, :]).reshape(D, H)
        return jnp.concatenate([bs, bd], axis=1)

    x_a = ablk(x_asrc, x_adst)
    m2_a = ablk(m2_asrc, m2_adst)
    m1_a = ablk(m1_asrc, m1_adst)

    vis3 = visual_feat.reshape(B, 1, Dv)
    aud3 = acoustic_feat.reshape(B, 1, Da)
    erc3 = erc_label.reshape(B, _N1, 1).astype(jnp.int32)
    ecpe3 = ecpe_label.reshape(B, _N2, 1).astype(jnp.int32)

    const = lambda shape: pl.BlockSpec(shape, lambda b: tuple(0 for _ in shape))
    _kernel_fn = functools.partial(_model_kernel, dh=dh, alpha=_ALPHA)
    out = pl.pallas_call(
        _kernel_fn,
        out_shape=jax.ShapeDtypeStruct((nblk, 1, 2), jnp.float32),
        grid=(nblk,),
        in_specs=[
            pl.BlockSpec((_BLK, _NTXT, D), lambda b: (b, 0, 0)),
            pl.BlockSpec((_BLK, 1, Dv), lambda b: (b, 0, 0)),
            pl.BlockSpec((_BLK, 1, Da), lambda b: (b, 0, 0)),
            const((Dv, D)), const((1, D)), const((Da, D)), const((1, D)),
            const((D, D)), const((D, 2 * H)), const((_NX, H * _GRP)),
            const((D, D)), const((D, 2 * H)), const((_N2, H * _GRP)),
            const((D, D)), const((D, 2 * H)), const((_N1, H * _GRP)),
            const((_TAPS * H, H * _GRP)), const((H * _GRP, H * _GRP)),
            const((_TAPS, H * _GRP, D)),
            const((D, D)), const((1, D)), const((D, C)), const((1, C)),
            pl.BlockSpec((_BLK, _N1, 1), lambda b: (b, 0, 0)),
            pl.BlockSpec((_BLK, _N2, 1), lambda b: (b, 0, 0)),
        ],
        out_specs=pl.BlockSpec((1, 1, 2), lambda b: (b, 0, 0),
                               memory_space=pltpu.MemorySpace.SMEM),
        compiler_params=pltpu.CompilerParams(
            dimension_semantics=("parallel",)),
    )(causal_text, vis3, aud3,
      vis_w, vis_b.reshape(1, D), aud_w, aud_b.reshape(1, D),
      x_W, x_a, jnp.asarray(_tap_mask(_NX, 1)),
      m2_W, m2_a, jnp.asarray(_tap_mask(_N2, 1)),
      m1_W, m1_a, jnp.asarray(_tap_mask(_N1, 1)),
      jnp.asarray(_expand_mat()), jnp.asarray(_group_sum_mat()),
      jnp.asarray(_bcast_mat(dh), jnp.bfloat16),
      cls_w1, cls_b1.reshape(1, D), cls_w2, cls_b2.reshape(1, C),
      erc3, ecpe3)

    emotion_loss = jnp.sum(out[:, 0, 0]) / float(B * _N1)
    cause_loss = jnp.sum(out[:, 0, 1]) / float(B * _N2)
    return emotion_loss, cause_loss
